# baseline (speedup 1.0000x reference)
import jax
import jax.numpy as jnp
from jax.experimental import pallas as pl
from jax.experimental.pallas import tpu as pltpu

_B = 16384
_BLK = 2048
_H = 5
_W = 11
_OH = 20
_OW = 11


def _dg(lhs, rhs, dims):
    return jax.lax.dot_general(lhs, rhs, dimension_numbers=(dims, ((), ())),
                               preferred_element_type=jnp.float32)


def _mega_kernel(xT_ref, eT_ref, W1t_ref, W2_ref, W3_ref, R1t_ref, R2_ref,
                 R3t_ref, b1r_ref, b2r_ref, b3r_ref, c1r_ref, c2r_ref,
                 c3r_ref, meanT_ref, stdeT_ref, outT_ref,
                 Gs_scr, a_scr, Cp_scr, dp_scr):
    @pl.when(pl.program_id(0) == 0)
    def _fold():
        G3 = _dg(W3_ref[...], W2_ref[...], ((1,), (0,)))
        G = _dg(G3, W1t_ref[...], ((1,), (1,)))
        for h in range(_H):
            Gs_scr[h] = G[:, _W * h:_W * (h + 1)]
        a_scr[...] = (jnp.dot(G3, b1r_ref[...], preferred_element_type=jnp.float32)
                      + jnp.dot(W3_ref[...], b2r_ref[...], preferred_element_type=jnp.float32)
                      + b3r_ref[...])
        Hm = _dg(R3t_ref[...], R2_ref[...], ((0,), (0,)))
        C = _dg(Hm, R1t_ref[...], ((1,), (1,)))
        d = (jnp.dot(Hm, c1r_ref[...], preferred_element_type=jnp.float32)
             + _dg(R3t_ref[...], c2r_ref[...], ((0,), (0,)))
             + c3r_ref[...])
        Cp_scr[...] = C.reshape(_OH, _OW).T[:, :, None]
        dp_scr[...] = d.reshape(_OH, _OW).T[:, :, None]

    f32 = jnp.float32
    acc = jnp.dot(Gs_scr[0], xT_ref[0], preferred_element_type=f32)
    for h in range(1, _H):
        acc += jnp.dot(Gs_scr[h], xT_ref[h], preferred_element_type=f32)
    z = acc + a_scr[...]
    mean = z[0:1, :]
    std = z[1:2, :]
    sampled = mean + std * eT_ref[...]
    meanT_ref[...] = mean
    stdeT_ref[...] = jnp.exp(std)
    outT_ref[...] = sampled[None] * Cp_scr[...] + dp_scr[...]


def kernel(x, W1, b1, W2, b2, W3, b3, R1, c1, R2, c2, R3, c3, e):
    f32 = jnp.float32
    xT = jnp.transpose(x, (1, 2, 0))
    eT = e.reshape(1, _B)
    W1t = W1.T
    R1t = R1.T
    R3t = R3.T

    grid = (_B // _BLK,)
    meanT, stdeT, outT = pl.pallas_call(
        _mega_kernel,
        grid=grid,
        in_specs=[
            pl.BlockSpec((_H, _W, _BLK), lambda i: (0, 0, i)),
            pl.BlockSpec((1, _BLK), lambda i: (0, i)),
            pl.BlockSpec((_H * _W, 64), lambda i: (0, 0)),
            pl.BlockSpec((64, 64), lambda i: (0, 0)),
            pl.BlockSpec((2, 64), lambda i: (0, 0)),
            pl.BlockSpec((1, 128), lambda i: (0, 0)),
            pl.BlockSpec((64, 128), lambda i: (0, 0)),
            pl.BlockSpec((64, _OH * _OW), lambda i: (0, 0)),
            pl.BlockSpec((64, 1), lambda i: (0, 0)),
            pl.BlockSpec((64, 1), lambda i: (0, 0)),
            pl.BlockSpec((2, 1), lambda i: (0, 0)),
            pl.BlockSpec((128, 1), lambda i: (0, 0)),
            pl.BlockSpec((64, 1), lambda i: (0, 0)),
            pl.BlockSpec((_OH * _OW, 1), lambda i: (0, 0)),
        ],
        out_specs=(
            pl.BlockSpec((1, _BLK), lambda i: (0, i)),
            pl.BlockSpec((1, _BLK), lambda i: (0, i)),
            pl.BlockSpec((_OW, _OH, _BLK), lambda i: (0, 0, i)),
        ),
        out_shape=(
            jax.ShapeDtypeStruct((1, _B), f32),
            jax.ShapeDtypeStruct((1, _B), f32),
            jax.ShapeDtypeStruct((_OW, _OH, _B), f32),
        ),
        scratch_shapes=[
            pltpu.VMEM((_H, 2, _W), f32),
            pltpu.VMEM((2, 1), f32),
            pltpu.VMEM((_OW, _OH, 1), f32),
            pltpu.VMEM((_OW, _OH, 1), f32),
        ],
    )(xT, eT, W1t, W2, W3, R1t, R2, R3t,
      b1[:, None], b2[:, None], b3[:, None], c1[:, None], c2[:, None],
      c3[:, None])

    mean = meanT.reshape(_B, 1)
    stde = stdeT.reshape(_B, 1)
    produced = jnp.transpose(outT, (2, 1, 0))
    return (mean, stde, produced)


# speedup vs baseline: 1.1074x; 1.1074x over previous
import jax
import jax.numpy as jnp
from jax.experimental import pallas as pl
from jax.experimental.pallas import tpu as pltpu

_B = 16384
_BLK = 8192
_H = 5
_W = 11
_OH = 20
_OW = 11


def _dg(lhs, rhs, dims):
    return jax.lax.dot_general(lhs, rhs, dimension_numbers=(dims, ((), ())),
                               preferred_element_type=jnp.float32)


def _mega_kernel(xT_ref, eT_ref, W1t_ref, W2_ref, W3_ref, R1t_ref, R2_ref,
                 R3t_ref, b1r_ref, b2r_ref, b3r_ref, c1r_ref, c2r_ref,
                 c3r_ref, meanT_ref, stdeT_ref, outT_ref,
                 Gs_scr, a_scr, Cp_scr, dp_scr):
    @pl.when(pl.program_id(0) == 0)
    def _fold():
        G3 = _dg(W3_ref[...], W2_ref[...], ((1,), (0,)))
        G = _dg(G3, W1t_ref[...], ((1,), (1,)))
        for h in range(_H):
            Gs_scr[h] = G[:, _W * h:_W * (h + 1)]
        a_scr[...] = (jnp.dot(G3, b1r_ref[...], preferred_element_type=jnp.float32)
                      + jnp.dot(W3_ref[...], b2r_ref[...], preferred_element_type=jnp.float32)
                      + b3r_ref[...])
        Hm = _dg(R3t_ref[...], R2_ref[...], ((0,), (0,)))
        C = _dg(Hm, R1t_ref[...], ((1,), (1,)))
        d = (jnp.dot(Hm, c1r_ref[...], preferred_element_type=jnp.float32)
             + _dg(R3t_ref[...], c2r_ref[...], ((0,), (0,)))
             + c3r_ref[...])
        Cp_scr[...] = C.reshape(_OH, _OW).T[:, :, None]
        dp_scr[...] = d.reshape(_OH, _OW).T[:, :, None]

    f32 = jnp.float32
    acc = jnp.dot(Gs_scr[0], xT_ref[0], preferred_element_type=f32)
    for h in range(1, _H):
        acc += jnp.dot(Gs_scr[h], xT_ref[h], preferred_element_type=f32)
    z = acc + a_scr[...]
    mean = z[0:1, :]
    std = z[1:2, :]
    sampled = mean + std * eT_ref[...]
    meanT_ref[...] = mean
    stdeT_ref[...] = jnp.exp(std)
    outT_ref[...] = sampled[None] * Cp_scr[...] + dp_scr[...]


def kernel(x, W1, b1, W2, b2, W3, b3, R1, c1, R2, c2, R3, c3, e):
    f32 = jnp.float32
    xT = jnp.transpose(x, (1, 2, 0))
    eT = e.reshape(1, _B)
    W1t = W1.T
    R1t = R1.T
    R3t = R3.T

    grid = (_B // _BLK,)
    meanT, stdeT, outT = pl.pallas_call(
        _mega_kernel,
        grid=grid,
        in_specs=[
            pl.BlockSpec((_H, _W, _BLK), lambda i: (0, 0, i)),
            pl.BlockSpec((1, _BLK), lambda i: (0, i)),
            pl.BlockSpec((_H * _W, 64), lambda i: (0, 0)),
            pl.BlockSpec((64, 64), lambda i: (0, 0)),
            pl.BlockSpec((2, 64), lambda i: (0, 0)),
            pl.BlockSpec((1, 128), lambda i: (0, 0)),
            pl.BlockSpec((64, 128), lambda i: (0, 0)),
            pl.BlockSpec((64, _OH * _OW), lambda i: (0, 0)),
            pl.BlockSpec((64, 1), lambda i: (0, 0)),
            pl.BlockSpec((64, 1), lambda i: (0, 0)),
            pl.BlockSpec((2, 1), lambda i: (0, 0)),
            pl.BlockSpec((128, 1), lambda i: (0, 0)),
            pl.BlockSpec((64, 1), lambda i: (0, 0)),
            pl.BlockSpec((_OH * _OW, 1), lambda i: (0, 0)),
        ],
        out_specs=(
            pl.BlockSpec((1, _BLK), lambda i: (0, i)),
            pl.BlockSpec((1, _BLK), lambda i: (0, i)),
            pl.BlockSpec((_OW, _OH, _BLK), lambda i: (0, 0, i)),
        ),
        out_shape=(
            jax.ShapeDtypeStruct((1, _B), f32),
            jax.ShapeDtypeStruct((1, _B), f32),
            jax.ShapeDtypeStruct((_OW, _OH, _B), f32),
        ),
        scratch_shapes=[
            pltpu.VMEM((_H, 2, _W), f32),
            pltpu.VMEM((2, 1), f32),
            pltpu.VMEM((_OW, _OH, 1), f32),
            pltpu.VMEM((_OW, _OH, 1), f32),
        ],
    )(xT, eT, W1t, W2, W3, R1t, R2, R3t,
      b1[:, None], b2[:, None], b3[:, None], c1[:, None], c2[:, None],
      c3[:, None])

    mean = meanT.reshape(_B, 1)
    stde = stdeT.reshape(_B, 1)
    produced = jnp.transpose(outT, (2, 1, 0))
    return (mean, stde, produced)
